# baseline (speedup 1.0000x reference)
import functools

import jax
import jax.numpy as jnp
from jax import lax
from jax.experimental import pallas as pl
from jax.experimental.pallas import tpu as pltpu
from jax.experimental.pallas import tpu_sc as plsc

EMBED = 64
BATCH = 16384
VOCAB = 100001
BMV = 2048

NC = 2
NS = 16
NW = NC * NS
PER_W = BATCH // NW
CHUNK = 128
NCHUNK = PER_W // CHUNK


def _pmat_body(tt_ref, wt_ref, b_ref, p_ref):
    tt = tt_ref[...]
    hb = 0.5 * b_ref[...]
    dn = (((0,), (1,)), ((), ()))
    p_ref[:, 0:EMBED] = lax.dot_general(
        tt, wt_ref[:, 0:EMBED], dn, preferred_element_type=jnp.float32) + hb
    p_ref[:, EMBED:2 * EMBED] = lax.dot_general(
        tt, wt_ref[:, EMBED:2 * EMBED], dn,
        preferred_element_type=jnp.float32) + hb


def _pmat(tableT, WT, b2d):
    return pl.pallas_call(
        _pmat_body,
        grid=(pl.cdiv(VOCAB, BMV),),
        in_specs=[
            pl.BlockSpec((EMBED, BMV), lambda i: (0, i)),
            pl.BlockSpec((EMBED, 2 * EMBED), lambda i: (0, 0)),
            pl.BlockSpec((1, EMBED), lambda i: (0, 0)),
        ],
        out_specs=pl.BlockSpec((BMV, 2 * EMBED), lambda i: (i, 0)),
        out_shape=jax.ShapeDtypeStruct((VOCAB, 2 * EMBED), jnp.float32),
        compiler_params=pltpu.CompilerParams(
            fuse_transposed_lhs_in_matmul=True),
    )(tableT, WT, b2d)


def _combine_chunk(gx_v, gy_v, buf, o_v):
    def body(r, _):
        for q in range(EMBED // 16):
            s = q * 16
            o_v[buf, r, pl.ds(s, 16)] = (
                gx_v[buf, r, pl.ds(s, 16)]
                + gy_v[buf, r, pl.ds(EMBED + s, 16)])
        return 0

    lax.fori_loop(0, CHUNK, body, 0, unroll=4)


@functools.partial(
    pl.kernel,
    mesh=plsc.VectorSubcoreMesh(core_axis_name="c", subcore_axis_name="s"),
    out_type=jax.ShapeDtypeStruct((BATCH, EMBED), jnp.float32),
    scratch_types=[
        pltpu.VMEM((NCHUNK, CHUNK), jnp.int32),
        pltpu.VMEM((NCHUNK, CHUNK), jnp.int32),
        pltpu.VMEM((2, CHUNK, 2 * EMBED), jnp.float32),
        pltpu.VMEM((2, CHUNK, 2 * EMBED), jnp.float32),
        pltpu.VMEM((2, CHUNK, EMBED), jnp.float32),
        pltpu.SemaphoreType.DMA,
        pltpu.SemaphoreType.DMA,
        pltpu.SemaphoreType.DMA,
    ],
    compiler_params=pltpu.CompilerParams(use_tc_tiling_on_sc=True),
)
def _gather_combine(idx_hbm, p_hbm, out_hbm, ix_v, iy_v, gx_v, gy_v, o_v,
                    isem, gsem, osem):
    wid = lax.axis_index("s") * NC + lax.axis_index("c")
    base = wid * PER_W
    icp = [
        pltpu.async_copy(idx_hbm.at[pl.ds(base + j * CHUNK, CHUNK)],
                         ix_v.at[j], isem)
        for j in range(NCHUNK)
    ] + [
        pltpu.async_copy(idx_hbm.at[pl.ds(BATCH + base + j * CHUNK, CHUNK)],
                         iy_v.at[j], isem)
        for j in range(NCHUNK)
    ]
    for c in icp:
        c.wait()

    gets = [(pltpu.async_copy(p_hbm.at[ix_v.at[0]], gx_v.at[0], gsem),
             pltpu.async_copy(p_hbm.at[iy_v.at[0]], gy_v.at[0], gsem))]
    puts = []
    for j in range(NCHUNK):
        buf = j % 2
        if j + 1 < NCHUNK:
            nb = (j + 1) % 2
            gets.append(
                (pltpu.async_copy(p_hbm.at[ix_v.at[j + 1]], gx_v.at[nb], gsem),
                 pltpu.async_copy(p_hbm.at[iy_v.at[j + 1]], gy_v.at[nb], gsem)))
        gets[j][0].wait()
        gets[j][1].wait()
        if j >= 2:
            puts[j - 2].wait()
        _combine_chunk(gx_v, gy_v, buf, o_v)
        puts.append(pltpu.async_copy(
            o_v.at[buf], out_hbm.at[pl.ds(base + j * CHUNK, CHUNK)], osem))
    puts[NCHUNK - 2].wait()
    puts[NCHUNK - 1].wait()


def kernel(actions, table, W, b):
    idx = actions.astype(jnp.int32).T.reshape(2 * BATCH)
    P = _pmat(table.T, W.T, b.reshape(1, EMBED))
    return _gather_combine(idx, P)


# speedup vs baseline: 1.1514x; 1.1514x over previous
import functools

import jax
import jax.numpy as jnp
from jax import lax
from jax.experimental import pallas as pl
from jax.experimental.pallas import tpu as pltpu
from jax.experimental.pallas import tpu_sc as plsc

EMBED = 64
BATCH = 16384
VOCAB = 100001
BMV = 4096

NC = 2
NS = 16
NW = NC * NS
PER_W = BATCH // NW
CHUNK = 128
NCHUNK = PER_W // CHUNK


def _pmat_body(tt_ref, wt_ref, b_ref, p_ref):
    tt = tt_ref[...]
    hb = 0.5 * b_ref[...]
    dn = (((0,), (1,)), ((), ()))
    p_ref[:, 0:EMBED] = lax.dot_general(
        tt, wt_ref[:, 0:EMBED], dn, preferred_element_type=jnp.float32) + hb
    p_ref[:, EMBED:2 * EMBED] = lax.dot_general(
        tt, wt_ref[:, EMBED:2 * EMBED], dn,
        preferred_element_type=jnp.float32) + hb


def _pmat(tableT, WT, b2d):
    return pl.pallas_call(
        _pmat_body,
        grid=(pl.cdiv(VOCAB, BMV),),
        in_specs=[
            pl.BlockSpec((EMBED, BMV), lambda i: (0, i)),
            pl.BlockSpec((EMBED, 2 * EMBED), lambda i: (0, 0)),
            pl.BlockSpec((1, EMBED), lambda i: (0, 0)),
        ],
        out_specs=pl.BlockSpec((BMV, 2 * EMBED), lambda i: (i, 0)),
        out_shape=jax.ShapeDtypeStruct((VOCAB, 2 * EMBED), jnp.float32),
    )(tableT, WT, b2d)


def _combine_chunk(gx_v, gy_v, buf, o_v):
    def body(r, _):
        for q in range(EMBED // 16):
            s = q * 16
            o_v[buf, r, pl.ds(s, 16)] = (
                gx_v[buf, r, pl.ds(s, 16)]
                + gy_v[buf, r, pl.ds(EMBED + s, 16)])
        return 0

    lax.fori_loop(0, CHUNK, body, 0, unroll=4)


@functools.partial(
    pl.kernel,
    mesh=plsc.VectorSubcoreMesh(core_axis_name="c", subcore_axis_name="s"),
    out_type=jax.ShapeDtypeStruct((BATCH, EMBED), jnp.float32),
    scratch_types=[
        pltpu.VMEM((NCHUNK, CHUNK), jnp.int32),
        pltpu.VMEM((NCHUNK, CHUNK), jnp.int32),
        pltpu.VMEM((2, CHUNK, 2 * EMBED), jnp.float32),
        pltpu.VMEM((2, CHUNK, 2 * EMBED), jnp.float32),
        pltpu.VMEM((2, CHUNK, EMBED), jnp.float32),
        pltpu.SemaphoreType.DMA,
        pltpu.SemaphoreType.DMA,
        pltpu.SemaphoreType.DMA,
    ],
    compiler_params=pltpu.CompilerParams(use_tc_tiling_on_sc=True),
)
def _gather_combine(idx_hbm, p_hbm, out_hbm, ix_v, iy_v, gx_v, gy_v, o_v,
                    isem, gsem, osem):
    wid = lax.axis_index("s") * NC + lax.axis_index("c")
    base = wid * PER_W
    icp = [
        pltpu.async_copy(idx_hbm.at[pl.ds(base + j * CHUNK, CHUNK)],
                         ix_v.at[j], isem)
        for j in range(NCHUNK)
    ] + [
        pltpu.async_copy(idx_hbm.at[pl.ds(BATCH + base + j * CHUNK, CHUNK)],
                         iy_v.at[j], isem)
        for j in range(NCHUNK)
    ]
    for c in icp:
        c.wait()

    gets = [(pltpu.async_copy(p_hbm.at[ix_v.at[0]], gx_v.at[0], gsem),
             pltpu.async_copy(p_hbm.at[iy_v.at[0]], gy_v.at[0], gsem))]
    puts = []
    for j in range(NCHUNK):
        buf = j % 2
        if j + 1 < NCHUNK:
            nb = (j + 1) % 2
            gets.append(
                (pltpu.async_copy(p_hbm.at[ix_v.at[j + 1]], gx_v.at[nb], gsem),
                 pltpu.async_copy(p_hbm.at[iy_v.at[j + 1]], gy_v.at[nb], gsem)))
        gets[j][0].wait()
        gets[j][1].wait()
        if j >= 2:
            puts[j - 2].wait()
        _combine_chunk(gx_v, gy_v, buf, o_v)
        puts.append(pltpu.async_copy(
            o_v.at[buf], out_hbm.at[pl.ds(base + j * CHUNK, CHUNK)], osem))
    puts[NCHUNK - 2].wait()
    puts[NCHUNK - 1].wait()


def kernel(actions, table, W, b):
    idx = actions.astype(jnp.int32).T.reshape(2 * BATCH)
    P = _pmat(table.T, W.T, b.reshape(1, EMBED))
    return _gather_combine(idx, P)
